# baseline (speedup 1.0000x reference)
import functools

import jax
import jax.numpy as jnp
from jax import lax
from jax.experimental import pallas as pl
from jax.experimental.pallas import tpu as pltpu
from jax.experimental.pallas import tpu_sc as plsc

HL = 112
H = 2 * HL
L = HL * HL
NC, NS = 2, 16
NW = NC * NS
G = 16
NCHUNK = H // G


def _sc_block_gather(planes, v3, pmap16):
    ppw = planes // NW

    mesh = plsc.VectorSubcoreMesh(core_axis_name="c", subcore_axis_name="s")

    @functools.partial(
        pl.kernel,
        out_type=jax.ShapeDtypeStruct((planes, H, H), jnp.float32),
        mesh=mesh,
        compiler_params=pltpu.CompilerParams(
            use_tc_tiling_on_sc=True, needs_layout_passes=False),
        scratch_types=[
            pltpu.VMEM((2, H, H), jnp.float32),
            pltpu.VMEM((L // 2,), jnp.int32),
            pltpu.VMEM((2, G, H), jnp.float32),
            pltpu.SemaphoreType.DMA,
            pltpu.SemaphoreType.DMA,
        ],
    )
    def k(v_hbm, pmap_hbm, out_hbm, vbuf, pbuf, obuf, vsem, osem):
        wid = lax.axis_index("s") * NC + lax.axis_index("c")
        b = wid // (NW // 2)
        p0 = wid * ppw
        iota = lax.iota(jnp.int32, 16)
        half_iota = iota >> 1
        idx_hi = 8 + half_iota
        even = (iota & 1) == 0
        gdn = lax.GatherDimensionNumbers(
            offset_dims=(), collapsed_slice_dims=(0,), start_index_map=(0,))

        def perm(v, idx):
            return lax.gather(v, idx[:, None], gdn, slice_sizes=(1,),
                              mode=lax.GatherScatterMode.PROMISE_IN_BOUNDS)

        pltpu.make_async_copy(v_hbm.at[p0], vbuf.at[0], vsem).start()
        pltpu.sync_copy(pmap_hbm.at[pl.ds(b * (L // 2), L // 2)], pbuf)

        def plane_body(p, carry):
            plane = p0 + p
            pb = p % 2
            pltpu.make_async_copy(v_hbm.at[plane], vbuf.at[pb], vsem).wait()

            @pl.when(p + 1 < ppw)
            def _prefetch_plane():
                pltpu.make_async_copy(
                    v_hbm.at[plane + 1], vbuf.at[1 - pb], vsem).start()

            vplane = vbuf.at[pb]

            def chunk_body(ck, carry2):
                cb = ck % 2
                row0 = ck * G

                @pl.when(ck >= 2)
                def _drain_prev_out():
                    pltpu.make_async_copy(
                        obuf.at[cb], out_hbm.at[0, pl.ds(0, G)], osem).wait()

                ochunk = obuf.at[cb]

                @plsc.parallel_loop(0, G // 4, unroll=2)
                def quad_body(q4):
                    off32 = (ck * (G // 2) + 2 * q4) * (HL // 2)
                    for m in range(7):
                        x = pbuf[pl.ds(off32 + 16 * m, 16)]
                        runs = (x & 0xFFFF, lax.shift_right_logical(x, 16))
                        for h in range(2):
                            r = 2 * m + h
                            pp, w16 = divmod(r, 7)
                            pm = runs[h]
                            srow = pm >> 8
                            scol = pm & 255
                            for i in range(2):
                                v0 = plsc.load_gather(
                                    vplane, [srow + i, scol])
                                v1 = plsc.load_gather(
                                    vplane, [srow + i, scol + 1])
                                out_a = jnp.where(
                                    even, perm(v0, half_iota),
                                    perm(v1, half_iota))
                                out_b = jnp.where(
                                    even, perm(v0, idx_hi), perm(v1, idx_hi))
                                row = 4 * q4 + 2 * pp + i
                                ochunk[row, pl.ds(32 * w16, 16)] = out_a
                                ochunk[row, pl.ds(32 * w16 + 16, 16)] = out_b

                pltpu.make_async_copy(
                    ochunk, out_hbm.at[plane, pl.ds(row0, G)], osem).start()
                return carry2

            lax.fori_loop(0, NCHUNK, chunk_body, 0)
            pltpu.make_async_copy(
                obuf.at[0], out_hbm.at[0, pl.ds(0, G)], osem).wait()
            pltpu.make_async_copy(
                obuf.at[0], out_hbm.at[0, pl.ds(0, G)], osem).wait()
            return carry

        lax.fori_loop(0, ppw, plane_body, 0)

    return k(v3, pmap16)


def kernel(lr, ref, index_map, value):
    B, C, Hv, Wv = value.shape
    im = index_map.astype(jnp.int32)
    pm = ((im // HL) * 2) * 256 + (im % HL) * 2
    pmr = pm.reshape(B, L // 32, 2, 16)
    pm16 = (pmr[:, :, 0, :] | (pmr[:, :, 1, :] << 16)).reshape(-1)
    v3 = value.reshape(B * C, Hv, Wv)
    out = _sc_block_gather(B * C, v3, pm16)
    return out.reshape(B, C, Hv, Wv)


# speedup vs baseline: 1.1183x; 1.1183x over previous
import functools

import jax
import jax.numpy as jnp
from jax import lax
from jax.experimental import pallas as pl
from jax.experimental.pallas import tpu as pltpu
from jax.experimental.pallas import tpu_sc as plsc

HL = 112
H = 2 * HL
L = HL * HL
NC, NS = 2, 16
NW = NC * NS
G = 16
NCHUNK = H // G


def _sc_block_gather(planes, v3, pmap16):
    ppw = planes // NW

    mesh = plsc.VectorSubcoreMesh(core_axis_name="c", subcore_axis_name="s")

    @functools.partial(
        pl.kernel,
        out_type=jax.ShapeDtypeStruct((planes, H, H), jnp.float32),
        mesh=mesh,
        compiler_params=pltpu.CompilerParams(
            use_tc_tiling_on_sc=True, needs_layout_passes=False),
        scratch_types=[
            pltpu.VMEM((2, H, H), jnp.float32),
            pltpu.VMEM((L // 2,), jnp.int32),
            pltpu.VMEM((2, G, H), jnp.float32),
            pltpu.SemaphoreType.DMA,
            pltpu.SemaphoreType.DMA,
        ],
    )
    def k(v_hbm, pmap_hbm, out_hbm, vbuf, pbuf, obuf, vsem, osem):
        wid = lax.axis_index("s") * NC + lax.axis_index("c")
        b = wid // (NW // 2)
        p0 = wid * ppw
        iota = lax.iota(jnp.int32, 16)
        half_iota = iota >> 1
        idx_hi = 8 + half_iota
        even = (iota & 1) == 0
        gdn = lax.GatherDimensionNumbers(
            offset_dims=(), collapsed_slice_dims=(0,), start_index_map=(0,))

        def perm(v, idx):
            return lax.gather(v, idx[:, None], gdn, slice_sizes=(1,),
                              mode=lax.GatherScatterMode.PROMISE_IN_BOUNDS)

        pltpu.make_async_copy(v_hbm.at[p0], vbuf.at[0], vsem).start()
        pltpu.sync_copy(pmap_hbm.at[pl.ds(b * (L // 2), L // 2)], pbuf)

        def plane_body(p, carry):
            plane = p0 + p
            pb = p % 2
            pltpu.make_async_copy(v_hbm.at[plane], vbuf.at[pb], vsem).wait()

            @pl.when(p + 1 < ppw)
            def _prefetch_plane():
                pltpu.make_async_copy(
                    v_hbm.at[plane + 1], vbuf.at[1 - pb], vsem).start()

            vplane = vbuf.at[pb]

            def chunk_body(ck, carry2):
                cb = ck % 2
                row0 = ck * G

                @pl.when(ck >= 2)
                def _drain_prev_out():
                    pltpu.make_async_copy(
                        obuf.at[cb], out_hbm.at[0, pl.ds(0, G)], osem).wait()

                ochunk = obuf.at[cb]

                @plsc.parallel_loop(0, G // 4, unroll=4)
                def quad_body(q4):
                    off32 = (ck * (G // 2) + 2 * q4) * (HL // 2)
                    for m in range(7):
                        x = pbuf[pl.ds(off32 + 16 * m, 16)]
                        runs = (x & 0xFFFF, lax.shift_right_logical(x, 16))
                        for h in range(2):
                            r = 2 * m + h
                            pp, w16 = divmod(r, 7)
                            pm = runs[h]
                            srow = pm >> 8
                            scol = pm & 255
                            for i in range(2):
                                v0 = plsc.load_gather(
                                    vplane, [srow + i, scol])
                                v1 = plsc.load_gather(
                                    vplane, [srow + i, scol + 1])
                                out_a = jnp.where(
                                    even, perm(v0, half_iota),
                                    perm(v1, half_iota))
                                out_b = jnp.where(
                                    even, perm(v0, idx_hi), perm(v1, idx_hi))
                                row = 4 * q4 + 2 * pp + i
                                ochunk[row, pl.ds(32 * w16, 16)] = out_a
                                ochunk[row, pl.ds(32 * w16 + 16, 16)] = out_b

                pltpu.make_async_copy(
                    ochunk, out_hbm.at[plane, pl.ds(row0, G)], osem).start()
                return carry2

            lax.fori_loop(0, NCHUNK, chunk_body, 0)
            pltpu.make_async_copy(
                obuf.at[0], out_hbm.at[0, pl.ds(0, G)], osem).wait()
            pltpu.make_async_copy(
                obuf.at[0], out_hbm.at[0, pl.ds(0, G)], osem).wait()
            return carry

        lax.fori_loop(0, ppw, plane_body, 0)

    return k(v3, pmap16)


def kernel(lr, ref, index_map, value):
    B, C, Hv, Wv = value.shape
    im = index_map.astype(jnp.int32)
    pm = ((im // HL) * 2) * 256 + (im % HL) * 2
    pmr = pm.reshape(B, L // 32, 2, 16)
    pm16 = (pmr[:, :, 0, :] | (pmr[:, :, 1, :] << 16)).reshape(-1)
    v3 = value.reshape(B * C, Hv, Wv)
    out = _sc_block_gather(B * C, v3, pm16)
    return out.reshape(B, C, Hv, Wv)
